# baseline (speedup 1.0000x reference)
import functools

import jax
import jax.numpy as jnp
from jax import lax
from jax.experimental import pallas as pl
from jax.experimental.pallas import tpu as pltpu
from jax.experimental.pallas import tpu_sc as plsc

BATCH = 16384
DIM = 32
LANES = 16
SUPER = 128
ROWS_PER_SUPER = SUPER // DIM

_info = plsc.get_sparse_core_info()
NC, NS = _info.num_cores, _info.num_subcores
NW = NC * NS
B_PER_W = BATCH // NW
CHUNK = 128
N_CHUNKS = B_PER_W // CHUNK
GROUPS_PER_CHUNK = CHUNK // LANES


def _gmf_body(users_hbm, items_hbm, ut_hbm, it_hbm, w_hbm, b_hbm, out_hbm,
              uidx_v, iidx_v, usup_v, isup_v,
              ubuf0, ubuf1, ibuf0, ibuf1, w_v, b_v, out_v, sem):
    wid = lax.axis_index("s") * NC + lax.axis_index("c")
    base = wid * B_PER_W

    pltpu.sync_copy(users_hbm.at[pl.ds(base, B_PER_W)], uidx_v)
    pltpu.sync_copy(items_hbm.at[pl.ds(base, B_PER_W)], iidx_v)
    pltpu.sync_copy(w_hbm, w_v)
    pltpu.sync_copy(b_hbm, b_v)

    def sup_body(j, _):
        sl = pl.ds(pl.multiple_of(j * LANES, LANES), LANES)
        usup_v[sl] = lax.shift_right_logical(uidx_v[sl], 2)
        isup_v[sl] = lax.shift_right_logical(iidx_v[sl], 2)
        return 0
    lax.fori_loop(0, B_PER_W // LANES, sup_body, 0)

    ubufs = (ubuf0, ubuf1)
    ibufs = (ibuf0, ibuf1)

    def fire(c):
        sl = pl.ds(c * CHUNK, CHUNK)
        return (pltpu.async_copy(ut_hbm.at[usup_v.at[sl]], ubufs[c % 2], sem),
                pltpu.async_copy(it_hbm.at[isup_v.at[sl]], ibufs[c % 2], sem))

    bias = b_v[...]
    lane = lax.broadcasted_iota(jnp.int32, (LANES,), 0)
    w_lo = w_v[pl.ds(0, LANES)]
    w_hi = w_v[pl.ds(LANES, LANES)]
    ws = [w_lo[k] for k in range(LANES)] + [w_hi[k] for k in range(LANES)]

    pending = fire(0)
    for c in range(N_CHUNKS):
        nxt = fire(c + 1) if c + 1 < N_CHUNKS else None
        pending[0].wait()
        pending[1].wait()
        ubuf, ibuf = ubufs[c % 2], ibufs[c % 2]

        def group_body(g, _, c=c, ubuf=ubuf, ibuf=ibuf):
            row0 = g * LANES
            gsl = pl.ds(c * CHUNK + row0, LANES)
            ucol0 = lax.shift_left(jnp.bitwise_and(uidx_v[gsl], 3), 5)
            icol0 = lax.shift_left(jnp.bitwise_and(iidx_v[gsl], 3), 5)
            rows = row0 + lane
            acc = bias
            for k in range(DIM):
                uk = plsc.load_gather(ubuf, [rows, ucol0 + k])
                ik = plsc.load_gather(ibuf, [rows, icol0 + k])
                acc = acc + uk * ik * ws[k]
            prob = 1.0 / (1.0 + jnp.exp(-acc))
            out_v[gsl] = prob
            return 0

        lax.fori_loop(0, GROUPS_PER_CHUNK, group_body, 0)
        pending = nxt

    pltpu.sync_copy(out_v, out_hbm.at[pl.ds(base, B_PER_W)])


@jax.jit
def _gmf_call(users, items, ut4, it4, w_flat, b_vec):
    mesh = plsc.VectorSubcoreMesh(core_axis_name="c", subcore_axis_name="s")
    kern = functools.partial(
        pl.kernel,
        out_type=jax.ShapeDtypeStruct((BATCH,), jnp.float32),
        mesh=mesh,
        compiler_params=pltpu.CompilerParams(needs_layout_passes=False),
        scratch_types=[
            pltpu.VMEM((B_PER_W,), jnp.int32),
            pltpu.VMEM((B_PER_W,), jnp.int32),
            pltpu.VMEM((B_PER_W,), jnp.int32),
            pltpu.VMEM((B_PER_W,), jnp.int32),
            pltpu.VMEM((CHUNK, SUPER), jnp.float32),
            pltpu.VMEM((CHUNK, SUPER), jnp.float32),
            pltpu.VMEM((CHUNK, SUPER), jnp.float32),
            pltpu.VMEM((CHUNK, SUPER), jnp.float32),
            pltpu.VMEM((DIM,), jnp.float32),
            pltpu.VMEM((LANES,), jnp.float32),
            pltpu.VMEM((B_PER_W,), jnp.float32),
            pltpu.SemaphoreType.DMA,
        ],
    )(_gmf_body)
    return kern(users, items, ut4, it4, w_flat, b_vec)


def kernel(users, items, user_table, item_table, W, b):
    ut4 = user_table.reshape(user_table.shape[0] // ROWS_PER_SUPER, SUPER)
    it4 = item_table.reshape(item_table.shape[0] // ROWS_PER_SUPER, SUPER)
    w_flat = W.reshape(DIM).astype(jnp.float32)
    b_vec = jnp.broadcast_to(b.reshape(()), (LANES,)).astype(jnp.float32)
    out = _gmf_call(users, items, ut4, it4, w_flat, b_vec)
    return out.reshape(BATCH, 1)
